# baseline (speedup 1.0000x reference)
import functools
import math

import numpy as np
import jax
import jax.numpy as jnp
from jax.experimental import pallas as pl
from jax.experimental.pallas import tpu as pltpu


def _interp_matrix(out_size, in_size, scale=1.0):
    m = np.zeros((out_size, in_size), dtype=np.float64)
    if out_size == 1 or in_size == 1:
        m[:, 0] = 1.0
        return (m * scale).astype(np.float32)
    src = np.arange(out_size, dtype=np.float64) * (in_size - 1) / (out_size - 1)
    lo = np.clip(np.floor(src).astype(np.int64), 0, in_size - 1)
    hi = np.minimum(lo + 1, in_size - 1)
    w = src - lo
    rows = np.arange(out_size)
    np.add.at(m, (rows, lo), 1.0 - w)
    np.add.at(m, (rows, hi), w)
    return (m * scale).astype(np.float32)


def _fused_body(x_ref, md_ref, mwT_ref, mh_ref, o_ref, *, do_block):
    x = x_ref[0, 0].astype(jnp.bfloat16)
    u = jax.lax.dot_general(md_ref[...], x, (((1,), (0,)), ((), ())),
                            preferred_element_type=jnp.float32)
    a = jax.lax.dot_general(u.astype(jnp.bfloat16), mwT_ref[...],
                            (((2,), (0,)), ((), ())),
                            preferred_element_type=jnp.float32)
    a = a.astype(jnp.bfloat16)
    mh = mh_ref[...]
    for b in range(do_block):
        o_ref[0, b] = jnp.dot(mh, a[b], preferred_element_type=jnp.float32)


@jax.jit
def kernel(x):
    N, C, D, H, W = x.shape
    factor = 2.0
    Do = int(math.floor(D * factor))
    Ho = int(math.floor(H * factor))
    Wo = int(math.floor(W * factor))

    md = jnp.asarray(_interp_matrix(Do, D), dtype=jnp.bfloat16)
    mh = jnp.asarray(_interp_matrix(Ho, H), dtype=jnp.bfloat16)
    mwT = jnp.asarray(np.ascontiguousarray(_interp_matrix(Wo, W, factor).T),
                      dtype=jnp.bfloat16)

    NC = N * C
    DoB = 64
    while Do % DoB:
        DoB //= 2
    T = Do // DoB

    out = pl.pallas_call(
        functools.partial(_fused_body, do_block=DoB),
        out_shape=jax.ShapeDtypeStruct((NC, Do, Ho, Wo), jnp.float32),
        grid=(1, NC * T),
        in_specs=[
            pl.BlockSpec((1, 1, D, H, W),
                         lambda i, t: (t // T // C, t // T % C, 0, 0, 0)),
            pl.BlockSpec((DoB, D), lambda i, t: (t % T, 0)),
            pl.BlockSpec((W, Wo), lambda i, t: (0, 0)),
            pl.BlockSpec((Ho, H), lambda i, t: (0, 0)),
        ],
        out_specs=pl.BlockSpec((1, DoB, Ho, Wo),
                               lambda i, t: (t // T, t % T, 0, 0)),
        compiler_params=pltpu.CompilerParams(
            dimension_semantics=("parallel", "parallel")),
    )(x, md, mwT, mh)

    return out.reshape(N, C, Do, Ho, Wo)


# speedup vs baseline: 1.0017x; 1.0017x over previous
import functools
import math

import numpy as np
import jax
import jax.numpy as jnp
from jax.experimental import pallas as pl
from jax.experimental.pallas import tpu as pltpu


def _interp_matrix(out_size, in_size, scale=1.0):
    m = np.zeros((out_size, in_size), dtype=np.float64)
    if out_size == 1 or in_size == 1:
        m[:, 0] = 1.0
        return (m * scale).astype(np.float32)
    src = np.arange(out_size, dtype=np.float64) * (in_size - 1) / (out_size - 1)
    lo = np.clip(np.floor(src).astype(np.int64), 0, in_size - 1)
    hi = np.minimum(lo + 1, in_size - 1)
    w = src - lo
    rows = np.arange(out_size)
    np.add.at(m, (rows, lo), 1.0 - w)
    np.add.at(m, (rows, hi), w)
    return (m * scale).astype(np.float32)


def _fused_body(x_ref, md_ref, mwT_ref, mh_ref, o_ref, *, do_block):
    x = x_ref[0, 0].astype(jnp.bfloat16)
    u = jax.lax.dot_general(md_ref[...], x, (((1,), (0,)), ((), ())),
                            preferred_element_type=jnp.float32)
    a = jax.lax.dot_general(u.astype(jnp.bfloat16), mwT_ref[...],
                            (((2,), (0,)), ((), ())),
                            preferred_element_type=jnp.float32)
    a = a.astype(jnp.bfloat16)
    mh = mh_ref[...]
    for b in range(do_block):
        o_ref[0, b] = jnp.dot(mh, a[b], preferred_element_type=jnp.float32)


@jax.jit
def kernel(x):
    N, C, D, H, W = x.shape
    factor = 2.0
    Do = int(math.floor(D * factor))
    Ho = int(math.floor(H * factor))
    Wo = int(math.floor(W * factor))

    md = jnp.asarray(_interp_matrix(Do, D), dtype=jnp.bfloat16)
    mh = jnp.asarray(_interp_matrix(Ho, H), dtype=jnp.bfloat16)
    mwT = jnp.asarray(np.ascontiguousarray(_interp_matrix(Wo, W, factor).T),
                      dtype=jnp.bfloat16)

    NC = N * C
    DoB = 64
    while Do % DoB:
        DoB //= 2
    T = Do // DoB

    out = pl.pallas_call(
        functools.partial(_fused_body, do_block=DoB),
        out_shape=jax.ShapeDtypeStruct((NC, Do, Ho, Wo), jnp.float32),
        grid=(1, NC * T),
        in_specs=[
            pl.BlockSpec((1, 1, D, H, W),
                         lambda i, t: (t // T // C, t // T % C, 0, 0, 0)),
            pl.BlockSpec((DoB, D), lambda i, t: (t % T, 0)),
            pl.BlockSpec((W, Wo), lambda i, t: (0, 0)),
            pl.BlockSpec((Ho, H), lambda i, t: (0, 0)),
        ],
        out_specs=pl.BlockSpec((1, DoB, Ho, Wo),
                               lambda i, t: (t // T, t % T, 0, 0)),
        compiler_params=pltpu.CompilerParams(
            dimension_semantics=("arbitrary", "arbitrary")),
    )(x, md, mwT, mh)

    return out.reshape(N, C, Do, Ho, Wo)
